# baseline (speedup 1.0000x reference)
import math
import functools

import jax
import jax.numpy as jnp
from jax.experimental import pallas as pl

B = 1
T = 2048
D_MODEL = 768
H = 12
DH = D_MODEL // H
P = 8
S = B * T * H

_INV_SQRT_DH = 1.0 / math.sqrt(DH)


def _compose_body(x_ref, w1c_ref, w2_ref, f_ref, e_ref,
                  pq_ref, gq_ref, pk_ref, gk_ref, pv_ref, gv_ref,
                  synq_ref, synk_ref, synv_ref,
                  logq_ref, logk_ref, logv_ref,
                  rawq_ref, rawk_ref, rawv_ref):
    xb = x_ref[...]
    fmat = f_ref[...]
    emat = e_ref[...]

    h_all = jnp.maximum(
        jax.lax.dot_general(xb, w1c_ref[...], (((1,), (0,)), ((), ())),
                            preferred_element_type=jnp.float32), 0.0)
    eo_parts = [
        jax.lax.dot_general(h_all[:, p * DH:(p + 1) * DH], w2_ref[p],
                            (((1,), (0,)), ((), ())),
                            preferred_element_type=jnp.float32)
        for p in range(P)
    ]
    eo_all = jnp.concatenate(eo_parts, axis=1)
    normT = jnp.sqrt(jax.lax.dot_general(
        fmat, eo_all * eo_all, (((0,), (1,)), ((), ())),
        preferred_element_type=jnp.float32))

    def one(p_ref, g_ref, syn_ref, log_ref, raw_ref):
        rawT = jax.lax.dot_general(p_ref[...], xb, (((1,), (1,)), ((), ())),
                                   preferred_element_type=jnp.float32)
        rawT = rawT * _INV_SQRT_DH - g_ref[...]
        logitT = jnp.maximum(rawT, 0.0)
        wT = jnp.where(logitT > 1e-6, logitT, 0.0)
        wrep = jax.lax.dot_general(wT, fmat, (((0,), (1,)), ((), ())),
                                   preferred_element_type=jnp.float32)
        syn_ref[...] = jax.lax.dot_general(
            eo_all * wrep, emat, (((1,), (0,)), ((), ())),
            preferred_element_type=jnp.float32)
        log_ref[...] = logitT
        raw_ref[...] = wT * normT

    one(pq_ref, gq_ref, synq_ref, logq_ref, rawq_ref)
    one(pk_ref, gk_ref, synk_ref, logk_ref, rawk_ref)
    one(pv_ref, gv_ref, synv_ref, logv_ref, rawv_ref)


def _compose(xf, w1cat, W2, fmat, emat,
             proto_q, gate_q, proto_k, gate_k, proto_v, gate_v, bs=3072):
    grid = (S // bs,)
    row = pl.BlockSpec((bs, DH), lambda i: (i, 0))
    small = pl.BlockSpec((P, bs), lambda i: (0, i))
    full = lambda shape: pl.BlockSpec(shape, lambda i: tuple(0 for _ in shape))
    out_shapes = (
        [jax.ShapeDtypeStruct((S, DH), jnp.float32)] * 3
        + [jax.ShapeDtypeStruct((P, S), jnp.float32)] * 6
    )
    return pl.pallas_call(
        _compose_body,
        grid=grid,
        in_specs=[row, full((DH, P * DH)), full((P, DH, DH)),
                  full((P * DH, P)), full((P * DH, DH)),
                  full((P, DH)), full((P, 1)),
                  full((P, DH)), full((P, 1)),
                  full((P, DH)), full((P, 1))],
        out_specs=[row, row, row, small, small, small, small, small, small],
        out_shape=out_shapes,
    )(xf, w1cat, W2, fmat, emat,
      proto_q, gate_q, proto_k, gate_k, proto_v, gate_v)


def _attn_body(q_ref, k_ref, v_ref, wot_ref, o_ref, *, bq, bk):
    i = pl.program_id(0)
    h = pl.program_id(1)
    q = q_ref[0]

    def body(j, carry):
        acc, m, l = carry
        kb = k_ref[0, pl.ds(j * bk, bk), :]
        vb = v_ref[0, pl.ds(j * bk, bk), :]
        s = jax.lax.dot_general(q, kb, (((1,), (1,)), ((), ())),
                                preferred_element_type=jnp.float32)
        s = s * _INV_SQRT_DH
        m_new = jnp.maximum(m, jnp.max(s, axis=1, keepdims=True))
        alpha = jnp.exp(m - m_new)
        pmat = jnp.exp(s - m_new)
        l = l * alpha + jnp.sum(pmat, axis=1, keepdims=True)
        acc = acc * alpha + jax.lax.dot_general(
            pmat, vb, (((1,), (0,)), ((), ())),
            preferred_element_type=jnp.float32)
        return acc, m_new, l

    nfull = (i * bq) // bk
    acc = jnp.zeros((bq, DH), jnp.float32)
    m0 = jnp.full((bq, 1), -jnp.inf, jnp.float32)
    l0 = jnp.zeros((bq, 1), jnp.float32)
    acc, m, l = jax.lax.fori_loop(0, nfull, body, (acc, m0, l0))

    kb = k_ref[0, pl.ds(i * bq, bq), :]
    vb = v_ref[0, pl.ds(i * bq, bq), :]
    s = jax.lax.dot_general(q, kb, (((1,), (1,)), ((), ())),
                            preferred_element_type=jnp.float32)
    s = s * _INV_SQRT_DH
    rows = jax.lax.broadcasted_iota(jnp.int32, (bq, bq), 0)
    cols = jax.lax.broadcasted_iota(jnp.int32, (bq, bq), 1)
    s = jnp.where(rows >= cols, s, -jnp.inf)
    m_new = jnp.maximum(m, jnp.max(s, axis=1, keepdims=True))
    alpha = jnp.exp(m - m_new)
    pmat = jnp.exp(s - m_new)
    l = l * alpha + jnp.sum(pmat, axis=1, keepdims=True)
    acc = acc * alpha + jax.lax.dot_general(
        pmat, vb, (((1,), (0,)), ((), ())),
        preferred_element_type=jnp.float32)
    attn = acc / l

    contrib = jax.lax.dot_general(attn, wot_ref[...], (((1,), (0,)), ((), ())),
                                  preferred_element_type=jnp.float32)

    @pl.when(h == 0)
    def _():
        o_ref[...] = contrib

    @pl.when(h != 0)
    def _():
        o_ref[...] += contrib


def _attention(q, k, v, WoT, bq=512, bk=512):
    grid = (T // bq, H)
    qspec = pl.BlockSpec((1, bq, DH), lambda i, h: (h, i, 0))
    kvspec = pl.BlockSpec((1, T, DH), lambda i, h: (h, 0, 0))
    return pl.pallas_call(
        functools.partial(_attn_body, bq=bq, bk=bk),
        grid=grid,
        in_specs=[qspec, kvspec, kvspec,
                  pl.BlockSpec((DH, D_MODEL), lambda i, h: (h, 0))],
        out_specs=pl.BlockSpec((bq, D_MODEL), lambda i, h: (i, 0)),
        out_shape=jax.ShapeDtypeStruct((T, D_MODEL), jnp.float32),
    )(q, k, v, WoT)


def _proj_body(x_ref, w_ref, o_ref):
    o_ref[...] = jax.lax.dot_general(x_ref[...], w_ref[...],
                                     (((1,), (1,)), ((), ())),
                                     preferred_element_type=jnp.float32)


def _out_proj(attn, Wo, br=512):
    grid = (T // br,)
    return pl.pallas_call(
        _proj_body,
        grid=grid,
        in_specs=[pl.BlockSpec((br, D_MODEL), lambda i: (i, 0)),
                  pl.BlockSpec((D_MODEL, D_MODEL), lambda i: (0, 0))],
        out_specs=pl.BlockSpec((br, D_MODEL), lambda i: (i, 0)),
        out_shape=jax.ShapeDtypeStruct((T, D_MODEL), jnp.float32),
    )(attn, Wo)


def kernel(x, position_ids, proto_q, gate_q, proto_k, gate_k, proto_v, gate_v,
           W1, W2, Wo):
    xf = x.reshape(S, DH)
    w1cat = jnp.transpose(W1, (1, 0, 2)).reshape(DH, P * DH)
    ridx = jnp.arange(P * DH, dtype=jnp.int32)
    fmat = (ridx[:, None] // DH == jnp.arange(P, dtype=jnp.int32)[None, :]
            ).astype(jnp.float32)
    emat = (ridx[:, None] % DH == jnp.arange(DH, dtype=jnp.int32)[None, :]
            ).astype(jnp.float32)

    (synq, synk, synv, logq, logk, logv, rawq, rawk, rawv) = _compose(
        xf, w1cat, W2, fmat, emat,
        proto_q, gate_q.reshape(P, 1),
        proto_k, gate_k.reshape(P, 1), proto_v, gate_v.reshape(P, 1))

    q = synq.reshape(T, H, DH).transpose(1, 0, 2)
    k = synk.reshape(T, H, DH).transpose(1, 0, 2)
    v = synv.reshape(T, H, DH).transpose(1, 0, 2)
    out = _attention(q, k, v, Wo.T).reshape(B, T, D_MODEL)

    shape_log = (B, T, H, P)
    return (out,
            logq.T.reshape(shape_log), logk.T.reshape(shape_log),
            logv.T.reshape(shape_log),
            rawq.T, rawk.T, rawv.T)


# speedup vs baseline: 1.0844x; 1.0844x over previous
import math
import functools

import jax
import jax.numpy as jnp
from jax.experimental import pallas as pl

B = 1
T = 2048
D_MODEL = 768
H = 12
DH = D_MODEL // H
P = 8
S = B * T * H

_INV_SQRT_DH = 1.0 / math.sqrt(DH)


def _compose_body(x_ref, w1c_ref, w2_ref, f_ref, e_ref,
                  pq_ref, gq_ref, pk_ref, gk_ref, pv_ref, gv_ref,
                  synq_ref, synk_ref, synv_ref,
                  logq_ref, logk_ref, logv_ref,
                  rawq_ref, rawk_ref, rawv_ref):
    xb = x_ref[...]
    fmat = f_ref[...]
    emat = e_ref[...]

    h_all = jnp.maximum(
        jax.lax.dot_general(xb, w1c_ref[...], (((1,), (0,)), ((), ())),
                            preferred_element_type=jnp.float32), 0.0)
    eo_parts = [
        jax.lax.dot_general(h_all[:, p * DH:(p + 1) * DH], w2_ref[p],
                            (((1,), (0,)), ((), ())),
                            preferred_element_type=jnp.float32)
        for p in range(P)
    ]
    eo_all = jnp.concatenate(eo_parts, axis=1)
    normT = jnp.sqrt(jax.lax.dot_general(
        fmat, eo_all * eo_all, (((0,), (1,)), ((), ())),
        preferred_element_type=jnp.float32))

    def one(p_ref, g_ref, syn_ref, log_ref, raw_ref):
        rawT = jax.lax.dot_general(p_ref[...], xb, (((1,), (1,)), ((), ())),
                                   preferred_element_type=jnp.float32)
        rawT = rawT * _INV_SQRT_DH - g_ref[...]
        logitT = jnp.maximum(rawT, 0.0)
        wT = jnp.where(logitT > 1e-6, logitT, 0.0)
        wrep = jax.lax.dot_general(wT, fmat, (((0,), (1,)), ((), ())),
                                   preferred_element_type=jnp.float32)
        syn_ref[...] = jax.lax.dot_general(
            eo_all * wrep, emat, (((1,), (0,)), ((), ())),
            preferred_element_type=jnp.float32)
        log_ref[...] = logitT
        raw_ref[...] = wT * normT

    one(pq_ref, gq_ref, synq_ref, logq_ref, rawq_ref)
    one(pk_ref, gk_ref, synk_ref, logk_ref, rawk_ref)
    one(pv_ref, gv_ref, synv_ref, logv_ref, rawv_ref)


def _compose(xf, w1cat, W2, fmat, emat,
             proto_q, gate_q, proto_k, gate_k, proto_v, gate_v, bs=3072):
    grid = (S // bs,)
    row = pl.BlockSpec((bs, DH), lambda i: (i, 0))
    small = pl.BlockSpec((P, bs), lambda i: (0, i))
    full = lambda shape: pl.BlockSpec(shape, lambda i: tuple(0 for _ in shape))
    out_shapes = (
        [jax.ShapeDtypeStruct((S, DH), jnp.float32)] * 3
        + [jax.ShapeDtypeStruct((P, S), jnp.float32)] * 6
    )
    return pl.pallas_call(
        _compose_body,
        grid=grid,
        in_specs=[row, full((DH, P * DH)), full((P, DH, DH)),
                  full((P * DH, P)), full((P * DH, DH)),
                  full((P, DH)), full((P, 1)),
                  full((P, DH)), full((P, 1)),
                  full((P, DH)), full((P, 1))],
        out_specs=[row, row, row, small, small, small, small, small, small],
        out_shape=out_shapes,
    )(xf, w1cat, W2, fmat, emat,
      proto_q, gate_q, proto_k, gate_k, proto_v, gate_v)


def _attn_body(q_ref, k_ref, v_ref, o_ref, *, bq, bk):
    i = pl.program_id(1)
    q = q_ref[0]

    def body(j, carry):
        acc, m, l = carry
        kb = k_ref[0, pl.ds(j * bk, bk), :]
        vb = v_ref[0, pl.ds(j * bk, bk), :]
        s = jax.lax.dot_general(q, kb, (((1,), (1,)), ((), ())),
                                preferred_element_type=jnp.float32)
        s = s * _INV_SQRT_DH
        m_new = jnp.maximum(m, jnp.max(s, axis=1, keepdims=True))
        alpha = jnp.exp(m - m_new)
        pmat = jnp.exp(s - m_new)
        l = l * alpha + jnp.sum(pmat, axis=1, keepdims=True)
        acc = acc * alpha + jax.lax.dot_general(
            pmat, vb, (((1,), (0,)), ((), ())),
            preferred_element_type=jnp.float32)
        return acc, m_new, l

    nfull = (i * bq) // bk
    acc = jnp.zeros((bq, DH), jnp.float32)
    m0 = jnp.full((bq, 1), -jnp.inf, jnp.float32)
    l0 = jnp.zeros((bq, 1), jnp.float32)
    acc, m, l = jax.lax.fori_loop(0, nfull, body, (acc, m0, l0))

    kb = k_ref[0, pl.ds(i * bq, bq), :]
    vb = v_ref[0, pl.ds(i * bq, bq), :]
    s = jax.lax.dot_general(q, kb, (((1,), (1,)), ((), ())),
                            preferred_element_type=jnp.float32)
    s = s * _INV_SQRT_DH
    rows = jax.lax.broadcasted_iota(jnp.int32, (bq, bq), 0)
    cols = jax.lax.broadcasted_iota(jnp.int32, (bq, bq), 1)
    s = jnp.where(rows >= cols, s, -jnp.inf)
    m_new = jnp.maximum(m, jnp.max(s, axis=1, keepdims=True))
    alpha = jnp.exp(m - m_new)
    pmat = jnp.exp(s - m_new)
    l = l * alpha + jnp.sum(pmat, axis=1, keepdims=True)
    acc = acc * alpha + jax.lax.dot_general(
        pmat, vb, (((1,), (0,)), ((), ())),
        preferred_element_type=jnp.float32)
    o_ref[0] = acc / l


def _attention(q, k, v, bq=1024, bk=1024):
    grid = (H, T // bq)
    qspec = pl.BlockSpec((1, bq, DH), lambda h, i: (h, i, 0))
    kvspec = pl.BlockSpec((1, T, DH), lambda h, i: (h, 0, 0))
    return pl.pallas_call(
        functools.partial(_attn_body, bq=bq, bk=bk),
        grid=grid,
        in_specs=[qspec, kvspec, kvspec],
        out_specs=qspec,
        out_shape=jax.ShapeDtypeStruct((H, T, DH), jnp.float32),
    )(q, k, v)


def _proj_body(x_ref, w_ref, o_ref):
    o_ref[...] = jax.lax.dot_general(x_ref[...], w_ref[...],
                                     (((1,), (1,)), ((), ())),
                                     preferred_element_type=jnp.float32)


def _out_proj(attn, Wo, br=512):
    grid = (T // br,)
    return pl.pallas_call(
        _proj_body,
        grid=grid,
        in_specs=[pl.BlockSpec((br, D_MODEL), lambda i: (i, 0)),
                  pl.BlockSpec((D_MODEL, D_MODEL), lambda i: (0, 0))],
        out_specs=pl.BlockSpec((br, D_MODEL), lambda i: (i, 0)),
        out_shape=jax.ShapeDtypeStruct((T, D_MODEL), jnp.float32),
    )(attn, Wo)


def kernel(x, position_ids, proto_q, gate_q, proto_k, gate_k, proto_v, gate_v,
           W1, W2, Wo):
    xf = x.reshape(S, DH)
    w1cat = jnp.transpose(W1, (1, 0, 2)).reshape(DH, P * DH)
    ridx = jnp.arange(P * DH, dtype=jnp.int32)
    fmat = (ridx[:, None] // DH == jnp.arange(P, dtype=jnp.int32)[None, :]
            ).astype(jnp.float32)
    emat = (ridx[:, None] % DH == jnp.arange(DH, dtype=jnp.int32)[None, :]
            ).astype(jnp.float32)

    (synq, synk, synv, logq, logk, logv, rawq, rawk, rawv) = _compose(
        xf, w1cat, W2, fmat, emat,
        proto_q, gate_q.reshape(P, 1),
        proto_k, gate_k.reshape(P, 1), proto_v, gate_v.reshape(P, 1))

    q = synq.reshape(T, H, DH).transpose(1, 0, 2)
    k = synk.reshape(T, H, DH).transpose(1, 0, 2)
    v = synv.reshape(T, H, DH).transpose(1, 0, 2)
    attn = _attention(q, k, v).transpose(1, 0, 2).reshape(T, D_MODEL)
    out = _out_proj(attn, Wo).reshape(B, T, D_MODEL)

    shape_log = (B, T, H, P)
    return (out,
            logq.T.reshape(shape_log), logk.T.reshape(shape_log),
            logv.T.reshape(shape_log),
            rawq.T, rawk.T, rawv.T)


# speedup vs baseline: 1.0975x; 1.0122x over previous
import math
import functools

import jax
import jax.numpy as jnp
from jax.experimental import pallas as pl

B = 1
T = 2048
D_MODEL = 768
H = 12
DH = D_MODEL // H
P = 8
S = B * T * H

_INV_SQRT_DH = 1.0 / math.sqrt(DH)


def _compose_body(x_ref, w1c_ref, w2_ref, f_ref, e_ref,
                  pq_ref, gq_ref, pk_ref, gk_ref, pv_ref, gv_ref,
                  synq_ref, synk_ref, synv_ref,
                  logq_ref, logk_ref, logv_ref,
                  rawq_ref, rawk_ref, rawv_ref):
    xb = x_ref[...]
    fmat = f_ref[...]
    emat = e_ref[...]

    h_all = jnp.maximum(
        jax.lax.dot_general(xb, w1c_ref[...], (((1,), (0,)), ((), ())),
                            preferred_element_type=jnp.float32), 0.0)
    eo_parts = [
        jax.lax.dot_general(h_all[:, p * DH:(p + 1) * DH], w2_ref[p],
                            (((1,), (0,)), ((), ())),
                            preferred_element_type=jnp.float32)
        for p in range(P)
    ]
    eo_all = jnp.concatenate(eo_parts, axis=1)
    normT = jnp.sqrt(jax.lax.dot_general(
        fmat, eo_all * eo_all, (((0,), (1,)), ((), ())),
        preferred_element_type=jnp.float32))

    def one(p_ref, g_ref, syn_ref, log_ref, raw_ref):
        rawT = jax.lax.dot_general(p_ref[...], xb, (((1,), (1,)), ((), ())),
                                   preferred_element_type=jnp.float32)
        rawT = rawT * _INV_SQRT_DH - g_ref[...]
        logitT = jnp.maximum(rawT, 0.0)
        wT = jnp.where(logitT > 1e-6, logitT, 0.0)
        wrep = jax.lax.dot_general(wT, fmat, (((0,), (1,)), ((), ())),
                                   preferred_element_type=jnp.float32)
        syn_ref[...] = jax.lax.dot_general(
            eo_all * wrep, emat, (((1,), (0,)), ((), ())),
            preferred_element_type=jnp.float32)
        log_ref[...] = logitT
        raw_ref[...] = wT * normT

    one(pq_ref, gq_ref, synq_ref, logq_ref, rawq_ref)
    one(pk_ref, gk_ref, synk_ref, logk_ref, rawk_ref)
    one(pv_ref, gv_ref, synv_ref, logv_ref, rawv_ref)


def _compose(xf, w1cat, W2, fmat, emat,
             proto_q, gate_q, proto_k, gate_k, proto_v, gate_v, bs=3072):
    grid = (S // bs,)
    row = pl.BlockSpec((bs, DH), lambda i: (i, 0))
    small = pl.BlockSpec((P, bs), lambda i: (0, i))
    full = lambda shape: pl.BlockSpec(shape, lambda i: tuple(0 for _ in shape))
    out_shapes = (
        [jax.ShapeDtypeStruct((S, DH), jnp.float32)] * 3
        + [jax.ShapeDtypeStruct((P, S), jnp.float32)] * 6
    )
    return pl.pallas_call(
        _compose_body,
        grid=grid,
        in_specs=[row, full((DH, P * DH)), full((P, DH, DH)),
                  full((P * DH, P)), full((P * DH, DH)),
                  full((P, DH)), full((P, 1)),
                  full((P, DH)), full((P, 1)),
                  full((P, DH)), full((P, 1))],
        out_specs=[row, row, row, small, small, small, small, small, small],
        out_shape=out_shapes,
    )(xf, w1cat, W2, fmat, emat,
      proto_q, gate_q, proto_k, gate_k, proto_v, gate_v)


def _attn_body(q_ref, k_ref, v_ref, o_ref, *, bq, bk):
    i = pl.program_id(1)
    q = q_ref[0]

    def body(j, carry):
        acc, m, l = carry
        kb = k_ref[0, pl.ds(j * bk, bk), :]
        vb = v_ref[0, pl.ds(j * bk, bk), :]
        s = jax.lax.dot_general(q, kb, (((1,), (1,)), ((), ())),
                                preferred_element_type=jnp.float32)
        s = s * _INV_SQRT_DH
        m_new = jnp.maximum(m, jnp.max(s, axis=1, keepdims=True))
        alpha = jnp.exp(m - m_new)
        pmat = jnp.exp(s - m_new)
        l = l * alpha + jnp.sum(pmat, axis=1, keepdims=True)
        acc = acc * alpha + jax.lax.dot_general(
            pmat, vb, (((1,), (0,)), ((), ())),
            preferred_element_type=jnp.float32)
        return acc, m_new, l

    nfull = (i * bq) // bk
    acc = jnp.zeros((bq, DH), jnp.float32)
    m0 = jnp.full((bq, 1), -jnp.inf, jnp.float32)
    l0 = jnp.zeros((bq, 1), jnp.float32)
    acc, m, l = jax.lax.fori_loop(0, nfull, body, (acc, m0, l0))

    kb = k_ref[0, pl.ds(i * bq, bq), :]
    vb = v_ref[0, pl.ds(i * bq, bq), :]
    s = jax.lax.dot_general(q, kb, (((1,), (1,)), ((), ())),
                            preferred_element_type=jnp.float32)
    s = s * _INV_SQRT_DH
    rows = jax.lax.broadcasted_iota(jnp.int32, (bq, bq), 0)
    cols = jax.lax.broadcasted_iota(jnp.int32, (bq, bq), 1)
    s = jnp.where(rows >= cols, s, -jnp.inf)
    m_new = jnp.maximum(m, jnp.max(s, axis=1, keepdims=True))
    alpha = jnp.exp(m - m_new)
    pmat = jnp.exp(s - m_new)
    l = l * alpha + jnp.sum(pmat, axis=1, keepdims=True)
    acc = acc * alpha + jax.lax.dot_general(
        pmat, vb, (((1,), (0,)), ((), ())),
        preferred_element_type=jnp.float32)
    o_ref[0] = acc / l


def _attention(q, k, v, bq=2048, bk=2048):
    grid = (H, T // bq)
    qspec = pl.BlockSpec((1, bq, DH), lambda h, i: (h, i, 0))
    kvspec = pl.BlockSpec((1, T, DH), lambda h, i: (h, 0, 0))
    return pl.pallas_call(
        functools.partial(_attn_body, bq=bq, bk=bk),
        grid=grid,
        in_specs=[qspec, kvspec, kvspec],
        out_specs=qspec,
        out_shape=jax.ShapeDtypeStruct((H, T, DH), jnp.float32),
    )(q, k, v)


def _proj_body(x_ref, w_ref, o_ref):
    o_ref[...] = jax.lax.dot_general(x_ref[...], w_ref[...],
                                     (((1,), (1,)), ((), ())),
                                     preferred_element_type=jnp.float32)


def _out_proj(attn, Wo, br=512):
    grid = (T // br,)
    return pl.pallas_call(
        _proj_body,
        grid=grid,
        in_specs=[pl.BlockSpec((br, D_MODEL), lambda i: (i, 0)),
                  pl.BlockSpec((D_MODEL, D_MODEL), lambda i: (0, 0))],
        out_specs=pl.BlockSpec((br, D_MODEL), lambda i: (i, 0)),
        out_shape=jax.ShapeDtypeStruct((T, D_MODEL), jnp.float32),
    )(attn, Wo)


def kernel(x, position_ids, proto_q, gate_q, proto_k, gate_k, proto_v, gate_v,
           W1, W2, Wo):
    xf = x.reshape(S, DH)
    w1cat = jnp.transpose(W1, (1, 0, 2)).reshape(DH, P * DH)
    ridx = jnp.arange(P * DH, dtype=jnp.int32)
    fmat = (ridx[:, None] // DH == jnp.arange(P, dtype=jnp.int32)[None, :]
            ).astype(jnp.float32)
    emat = (ridx[:, None] % DH == jnp.arange(DH, dtype=jnp.int32)[None, :]
            ).astype(jnp.float32)

    (synq, synk, synv, logq, logk, logv, rawq, rawk, rawv) = _compose(
        xf, w1cat, W2, fmat, emat,
        proto_q, gate_q.reshape(P, 1),
        proto_k, gate_k.reshape(P, 1), proto_v, gate_v.reshape(P, 1))

    q = synq.reshape(T, H, DH).transpose(1, 0, 2)
    k = synk.reshape(T, H, DH).transpose(1, 0, 2)
    v = synv.reshape(T, H, DH).transpose(1, 0, 2)
    attn = _attention(q, k, v).transpose(1, 0, 2).reshape(T, D_MODEL)
    out = _out_proj(attn, Wo).reshape(B, T, D_MODEL)

    shape_log = (B, T, H, P)
    return (out,
            logq.T.reshape(shape_log), logk.T.reshape(shape_log),
            logv.T.reshape(shape_log),
            rawq.T, rawk.T, rawv.T)
